# baseline (speedup 1.0000x reference)
import functools

import jax
import jax.numpy as jnp
from jax import lax
from jax.experimental import pallas as pl
from jax.experimental.pallas import tpu as pltpu
from jax.experimental.pallas import tpu_sc as plsc

_NB = 20000
_PRE = 12000
_POST = 2000
_THR = 0.7
_MIN = 16.0
_IMW = 800.0
_IMH = 800.0

_B = 1024
_T = 12
_NPAD = _T * _B

_ROWW = 16
_SCRAP = _NPAD
_BUFR = 2 * _NPAD

_HI = lax.Precision.HIGHEST


def _nms_kernel(x1_ref, y1_ref, x2_ref, y2_ref, tgt_out,
                ident_s, m_s, tri_s, area_s, krows_s, cnt_s):
    i = pl.program_id(0)

    @pl.when(i == 0)
    def _init():
        r = lax.broadcasted_iota(jnp.int32, (_B, _B), 0)
        c = lax.broadcasted_iota(jnp.int32, (_B, _B), 1)
        ident_s[:] = (r == c).astype(jnp.float32)
        tri_s[:] = (r <= c).astype(jnp.float32)
        area_s[:] = (x2_ref[:] - x1_ref[:] + 1.0) * (y2_ref[:] - y1_ref[:] + 1.0)
        cnt_s[0] = 0.0

    done = cnt_s[0] >= float(_POST)
    gpos = (lax.broadcasted_iota(jnp.int32, (1, _B), 1) + i * _B).astype(jnp.float32)

    @pl.when(jnp.logical_not(done))
    def _tile():
        def _t_col(row):
            return lax.dot_general(ident_s[:], row, (((1,), (1,)), ((), ())),
                                   preferred_element_type=jnp.float32,
                                   precision=_HI)

        cx1 = _t_col(x1_ref[pl.ds(i, 1), :])
        cy1 = _t_col(y1_ref[pl.ds(i, 1), :])
        cx2 = _t_col(x2_ref[pl.ds(i, 1), :])
        cy2 = _t_col(y2_ref[pl.ds(i, 1), :])
        carea = (cx2 - cx1 + 1.0) * (cy2 - cy1 + 1.0)
        jidx = lax.broadcasted_iota(jnp.int32, (_B, 1), 0)
        galive = ((i * _B + jidx) < _PRE).astype(jnp.float32)

        def _ovr_row(t):
            rx1 = x1_ref[pl.ds(t, 1), :]
            ry1 = y1_ref[pl.ds(t, 1), :]
            rx2 = x2_ref[pl.ds(t, 1), :]
            ry2 = y2_ref[pl.ds(t, 1), :]
            rarea = area_s[pl.ds(t, 1), :]
            w = jnp.maximum(0.0, jnp.minimum(cx2, rx2) - jnp.maximum(cx1, rx1) + 1.0)
            h = jnp.maximum(0.0, jnp.minimum(cy2, ry2) - jnp.maximum(cy1, ry1) + 1.0)
            inter = w * h
            return inter / (carea + rarea - inter)

        def _tbody(t, sup):
            flag = (_ovr_row(t) > _THR).astype(jnp.float32)
            krow = krows_s[pl.ds(t, 1), :]
            return sup + jnp.sum(flag * krow, axis=1, keepdims=True)

        sup0 = lax.fori_loop(0, i, _tbody, jnp.zeros((_B, 1), jnp.float32))
        alive = jnp.where(sup0 > 0.5, 0.0, galive)

        kidx = lax.broadcasted_iota(jnp.int32, (1, _B), 1)
        m_s[:] = ((_ovr_row(i) > _THR) & (jidx > kidx)).astype(jnp.float32)

        def _cond(c):
            return c[1]

        def _body(c):
            k, _ = c
            sup = lax.dot_general(m_s[:], k, (((1,), (0,)), ((), ())),
                                  preferred_element_type=jnp.float32,
                                  precision=_HI)
            nk = jnp.where(sup > 0.5, 0.0, alive)
            return nk, jnp.sum(jnp.abs(nk - k)) > 0.0

        keep_t, _ = lax.while_loop(_cond, _body, (alive, True))

        trow = lax.dot_general(keep_t, ident_s[:], (((0,), (0,)), ((), ())),
                               preferred_element_type=jnp.float32, precision=_HI)
        krows_s[pl.ds(i, 1), :] = trow
        prefix = lax.dot_general(trow, tri_s[:], (((1,), (0,)), ((), ())),
                                 preferred_element_type=jnp.float32,
                                 precision=_HI)
        tgt = jnp.where(trow > 0.5, cnt_s[0] + prefix - 1.0, float(_SCRAP) + gpos)
        tgt_out[pl.ds(i, 1), :] = tgt.astype(jnp.int32)
        cnt_s[0] = cnt_s[0] + jnp.sum(keep_t)

    @pl.when(done)
    def _skip():
        tgt_out[pl.ds(i, 1), :] = (float(_SCRAP) + gpos).astype(jnp.int32)


def _nms_targets(x1, y1, x2, y2):
    return pl.pallas_call(
        _nms_kernel,
        grid=(_T,),
        in_specs=[pl.BlockSpec((_T, _B), lambda i: (0, 0))] * 4,
        out_specs=pl.BlockSpec((_T, _B), lambda i: (0, 0)),
        out_shape=jax.ShapeDtypeStruct((_T, _B), jnp.int32),
        scratch_shapes=[
            pltpu.VMEM((_B, _B), jnp.float32),
            pltpu.VMEM((_B, _B), jnp.float32),
            pltpu.VMEM((_B, _B), jnp.float32),
            pltpu.VMEM((_T, _B), jnp.float32),
            pltpu.VMEM((_T, _B), jnp.float32),
            pltpu.SMEM((1,), jnp.float32),
        ],
    )(x1, y1, x2, y2)


def _sc_compact(props16, tgt):
    info = plsc.get_sparse_core_info()
    nw = info.num_cores * info.num_subcores
    per = _NPAD // nw
    mesh = plsc.VectorSubcoreMesh(core_axis_name="c", subcore_axis_name="s")

    @functools.partial(
        pl.kernel, mesh=mesh,
        compiler_params=pltpu.CompilerParams(use_tc_tiling_on_sc=False),
        out_type=jax.ShapeDtypeStruct((_BUFR, _ROWW), jnp.float32),
        scratch_types=[
            pltpu.VMEM((per,), jnp.int32),
            pltpu.VMEM((per, _ROWW), jnp.float32),
            pltpu.SemaphoreType.DMA,
        ],
    )
    def k(props_hbm, tgt_hbm, out_hbm, idx_v, rows_v, sem):
        wid = lax.axis_index("s") * info.num_cores + lax.axis_index("c")
        base = wid * per
        pltpu.sync_copy(tgt_hbm.at[pl.ds(base, per)], idx_v)
        pltpu.sync_copy(props_hbm.at[pl.ds(base, per)], rows_v)
        pltpu.async_copy(rows_v, out_hbm.at[idx_v], sem).wait()

    return k(props16, tgt)


def kernel(boxes, scores):
    x1 = jnp.clip(boxes[:, 0], 0.0, _IMW - 1.0)
    y1 = jnp.clip(boxes[:, 1], 0.0, _IMH - 1.0)
    x2 = jnp.clip(boxes[:, 2], 0.0, _IMW - 1.0)
    y2 = jnp.clip(boxes[:, 3], 0.0, _IMH - 1.0)
    ws = x2 - x1 + 1.0
    hs = y2 - y1 + 1.0
    size_ok = (ws >= _MIN) & (hs >= _MIN)
    sc = jnp.where(size_ok, scores, -jnp.inf)

    _, x1s, y1s, x2s, y2s, scs = lax.sort(
        (-sc, x1, y1, x2, y2, sc), dimension=0, num_keys=1, is_stable=True)

    pad = _NPAD - _PRE

    def _prep(a):
        return jnp.concatenate([a[:_PRE], jnp.zeros((pad,), a.dtype)]).reshape(_T, _B)

    tgt = _nms_targets(_prep(x1s), _prep(y1s), _prep(x2s), _prep(y2s))

    props16 = jnp.zeros((_NPAD, _ROWW), jnp.float32)
    props16 = props16.at[:, 0].set(x1s[:_NPAD])
    props16 = props16.at[:, 1].set(y1s[:_NPAD])
    props16 = props16.at[:, 2].set(x2s[:_NPAD])
    props16 = props16.at[:, 3].set(y2s[:_NPAD])
    props16 = props16.at[:, 4].set(scs[:_NPAD])

    buf = _sc_compact(props16, tgt.reshape(-1))

    cnt = jnp.sum((tgt.reshape(-1) < _POST).astype(jnp.int32))
    rows_ok = (jnp.arange(_POST) < cnt)[:, None]
    return jnp.where(rows_ok, buf[:_POST, :5], 0.0)


# speedup vs baseline: 1.7821x; 1.7821x over previous
import functools

import jax
import jax.numpy as jnp
from jax import lax
from jax.experimental import pallas as pl
from jax.experimental.pallas import tpu as pltpu
from jax.experimental.pallas import tpu_sc as plsc

_NB = 20000
_PRE = 12000
_POST = 2000
_THR = 0.7
_MIN = 16.0
_IMW = 800.0
_IMH = 800.0

_B = 1024
_T = 12
_NPAD = _T * _B

_ROWW = 8
_SCRAP = 2048
_BUFR = _SCRAP + _NPAD

_HI = lax.Precision.HIGHEST


def _nms_kernel(x1_ref, y1_ref, x2_ref, y2_ref, tgt_out,
                ident_s, m_s, tri_s, area_s, krows_s, cnt_s):
    i = pl.program_id(0)

    @pl.when(i == 0)
    def _init():
        r = lax.broadcasted_iota(jnp.int32, (_B, _B), 0)
        c = lax.broadcasted_iota(jnp.int32, (_B, _B), 1)
        ident_s[:] = (r == c).astype(jnp.float32)
        tri_s[:] = (r <= c).astype(jnp.float32)
        area_s[:] = (x2_ref[:] - x1_ref[:] + 1.0) * (y2_ref[:] - y1_ref[:] + 1.0)
        cnt_s[0] = 0.0

    done = cnt_s[0] >= float(_POST)
    gpos = (lax.broadcasted_iota(jnp.int32, (1, _B), 1) + i * _B).astype(jnp.float32)

    @pl.when(jnp.logical_not(done))
    def _tile():
        def _t_col(row):
            return lax.dot_general(ident_s[:], row, (((1,), (1,)), ((), ())),
                                   preferred_element_type=jnp.float32,
                                   precision=_HI)

        cx1 = _t_col(x1_ref[pl.ds(i, 1), :])
        cy1 = _t_col(y1_ref[pl.ds(i, 1), :])
        cx2 = _t_col(x2_ref[pl.ds(i, 1), :])
        cy2 = _t_col(y2_ref[pl.ds(i, 1), :])
        carea = (cx2 - cx1 + 1.0) * (cy2 - cy1 + 1.0)
        jidx = lax.broadcasted_iota(jnp.int32, (_B, 1), 0)
        galive = ((i * _B + jidx) < _PRE).astype(jnp.float32)

        def _ovr_row(t):
            rx1 = x1_ref[pl.ds(t, 1), :]
            ry1 = y1_ref[pl.ds(t, 1), :]
            rx2 = x2_ref[pl.ds(t, 1), :]
            ry2 = y2_ref[pl.ds(t, 1), :]
            rarea = area_s[pl.ds(t, 1), :]
            w = jnp.maximum(0.0, jnp.minimum(cx2, rx2) - jnp.maximum(cx1, rx1) + 1.0)
            h = jnp.maximum(0.0, jnp.minimum(cy2, ry2) - jnp.maximum(cy1, ry1) + 1.0)
            inter = w * h
            return inter / (carea + rarea - inter)

        def _tbody(t, sup):
            flag = (_ovr_row(t) > _THR).astype(jnp.float32)
            krow = krows_s[pl.ds(t, 1), :]
            return sup + jnp.sum(flag * krow, axis=1, keepdims=True)

        sup0 = lax.fori_loop(0, i, _tbody, jnp.zeros((_B, 1), jnp.float32))
        alive = jnp.where(sup0 > 0.5, 0.0, galive)

        kidx = lax.broadcasted_iota(jnp.int32, (1, _B), 1)
        m_s[:] = ((_ovr_row(i) > _THR) & (jidx > kidx)).astype(jnp.float32)

        def _cond(c):
            return c[1]

        def _body(c):
            k, _ = c
            sup = lax.dot_general(m_s[:], k, (((1,), (0,)), ((), ())),
                                  preferred_element_type=jnp.float32,
                                  precision=_HI)
            nk = jnp.where(sup > 0.5, 0.0, alive)
            return nk, jnp.sum(jnp.abs(nk - k)) > 0.0

        keep_t, _ = lax.while_loop(_cond, _body, (alive, True))

        trow = lax.dot_general(keep_t, ident_s[:], (((0,), (0,)), ((), ())),
                               preferred_element_type=jnp.float32, precision=_HI)
        krows_s[pl.ds(i, 1), :] = trow
        prefix = lax.dot_general(trow, tri_s[:], (((1,), (0,)), ((), ())),
                                 preferred_element_type=jnp.float32,
                                 precision=_HI)
        tgt = jnp.where(trow > 0.5, cnt_s[0] + prefix - 1.0, float(_SCRAP) + gpos)
        tgt_out[pl.ds(i, 1), :] = tgt.astype(jnp.int32)
        cnt_s[0] = cnt_s[0] + jnp.sum(keep_t)

    @pl.when(done)
    def _skip():
        tgt_out[pl.ds(i, 1), :] = (float(_SCRAP) + gpos).astype(jnp.int32)


def _nms_targets(x1, y1, x2, y2):
    return pl.pallas_call(
        _nms_kernel,
        grid=(_T,),
        in_specs=[pl.BlockSpec((_T, _B), lambda i: (0, 0))] * 4,
        out_specs=pl.BlockSpec((_T, _B), lambda i: (0, 0)),
        out_shape=jax.ShapeDtypeStruct((_T, _B), jnp.int32),
        scratch_shapes=[
            pltpu.VMEM((_B, _B), jnp.float32),
            pltpu.VMEM((_B, _B), jnp.float32),
            pltpu.VMEM((_B, _B), jnp.float32),
            pltpu.VMEM((_T, _B), jnp.float32),
            pltpu.VMEM((_T, _B), jnp.float32),
            pltpu.SMEM((1,), jnp.float32),
        ],
    )(x1, y1, x2, y2)


def _sc_compact(props16, tgt):
    info = plsc.get_sparse_core_info()
    nw = info.num_cores * info.num_subcores
    per = _NPAD // nw
    mesh = plsc.VectorSubcoreMesh(core_axis_name="c", subcore_axis_name="s")

    @functools.partial(
        pl.kernel, mesh=mesh,
        compiler_params=pltpu.CompilerParams(use_tc_tiling_on_sc=False),
        out_type=jax.ShapeDtypeStruct((_BUFR, _ROWW), jnp.float32),
        scratch_types=[
            pltpu.VMEM((per,), jnp.int32),
            pltpu.VMEM((per, _ROWW), jnp.float32),
            pltpu.SemaphoreType.DMA,
        ],
    )
    def k(props_hbm, tgt_hbm, out_hbm, idx_v, rows_v, sem):
        wid = lax.axis_index("s") * info.num_cores + lax.axis_index("c")
        base = wid * per
        pltpu.sync_copy(tgt_hbm.at[pl.ds(base, per)], idx_v)
        pltpu.sync_copy(props_hbm.at[pl.ds(base, per)], rows_v)
        pltpu.async_copy(rows_v, out_hbm.at[idx_v], sem).wait()

    return k(props16, tgt)


def kernel(boxes, scores):
    x1 = jnp.clip(boxes[:, 0], 0.0, _IMW - 1.0)
    y1 = jnp.clip(boxes[:, 1], 0.0, _IMH - 1.0)
    x2 = jnp.clip(boxes[:, 2], 0.0, _IMW - 1.0)
    y2 = jnp.clip(boxes[:, 3], 0.0, _IMH - 1.0)
    ws = x2 - x1 + 1.0
    hs = y2 - y1 + 1.0
    size_ok = (ws >= _MIN) & (hs >= _MIN)
    sc = jnp.where(size_ok, scores, -jnp.inf)

    _, x1s, y1s, x2s, y2s, scs = lax.sort(
        (-sc, x1, y1, x2, y2, sc), dimension=0, num_keys=1, is_stable=True)

    pad = _NPAD - _PRE

    def _prep(a):
        return jnp.concatenate([a[:_PRE], jnp.zeros((pad,), a.dtype)]).reshape(_T, _B)

    tgt = _nms_targets(_prep(x1s), _prep(y1s), _prep(x2s), _prep(y2s))

    props = jnp.concatenate(
        [x1s[:_NPAD, None], y1s[:_NPAD, None], x2s[:_NPAD, None],
         y2s[:_NPAD, None], scs[:_NPAD, None],
         jnp.zeros((_NPAD, _ROWW - 5), jnp.float32)], axis=1)

    buf = _sc_compact(props, tgt.reshape(-1))

    cnt = jnp.sum((tgt.reshape(-1) < _POST).astype(jnp.int32))
    rows_ok = (jnp.arange(_POST) < cnt)[:, None]
    return jnp.where(rows_ok, buf[:_POST, :5], 0.0)


# speedup vs baseline: 1.7825x; 1.0002x over previous
import functools

import jax
import jax.numpy as jnp
from jax import lax
from jax.experimental import pallas as pl
from jax.experimental.pallas import tpu as pltpu
from jax.experimental.pallas import tpu_sc as plsc

_NB = 20000
_PRE = 12000
_POST = 2000
_THR = 0.7
_MIN = 16.0
_IMW = 800.0
_IMH = 800.0

_B = 1024
_T = 12
_NPAD = _T * _B

_ROWW = 8
_SCRAP = 2048
_BUFR = _SCRAP + _NPAD

_HI = lax.Precision.HIGHEST


def _nms_kernel(x1_ref, y1_ref, x2_ref, y2_ref, tgt_out,
                ident_s, m_s, tri_s, area_s, krows_s, cnt_s):
    i = pl.program_id(0)

    @pl.when(i == 0)
    def _init():
        r = lax.broadcasted_iota(jnp.int32, (_B, _B), 0)
        c = lax.broadcasted_iota(jnp.int32, (_B, _B), 1)
        ident_s[:] = (r == c).astype(jnp.float32)
        tri_s[:] = (r <= c).astype(jnp.float32)
        area_s[:] = (x2_ref[:] - x1_ref[:] + 1.0) * (y2_ref[:] - y1_ref[:] + 1.0)
        cnt_s[0] = 0.0

    done = cnt_s[0] >= float(_POST)
    gpos = (lax.broadcasted_iota(jnp.int32, (1, _B), 1) + i * _B).astype(jnp.float32)

    @pl.when(jnp.logical_not(done))
    def _tile():
        def _t_col(row):
            return lax.dot_general(ident_s[:], row, (((1,), (1,)), ((), ())),
                                   preferred_element_type=jnp.float32,
                                   precision=_HI)

        cx1 = _t_col(x1_ref[pl.ds(i, 1), :])
        cy1 = _t_col(y1_ref[pl.ds(i, 1), :])
        cx2 = _t_col(x2_ref[pl.ds(i, 1), :])
        cy2 = _t_col(y2_ref[pl.ds(i, 1), :])
        carea = (cx2 - cx1 + 1.0) * (cy2 - cy1 + 1.0)
        jidx = lax.broadcasted_iota(jnp.int32, (_B, 1), 0)
        galive = ((i * _B + jidx) < _PRE).astype(jnp.float32)

        def _ovr_row(t):
            rx1 = x1_ref[pl.ds(t, 1), :]
            ry1 = y1_ref[pl.ds(t, 1), :]
            rx2 = x2_ref[pl.ds(t, 1), :]
            ry2 = y2_ref[pl.ds(t, 1), :]
            rarea = area_s[pl.ds(t, 1), :]
            w = jnp.maximum(0.0, jnp.minimum(cx2, rx2) - jnp.maximum(cx1, rx1) + 1.0)
            h = jnp.maximum(0.0, jnp.minimum(cy2, ry2) - jnp.maximum(cy1, ry1) + 1.0)
            inter = w * h
            return inter / (carea + rarea - inter)

        def _tbody(t, sup):
            flag = (_ovr_row(t) > _THR).astype(jnp.float32)
            krow = krows_s[pl.ds(t, 1), :]
            return sup + jnp.sum(flag * krow, axis=1, keepdims=True)

        sup0 = lax.fori_loop(0, i, _tbody, jnp.zeros((_B, 1), jnp.float32))
        alive = jnp.where(sup0 > 0.5, 0.0, galive)

        kidx = lax.broadcasted_iota(jnp.int32, (1, _B), 1)
        m_s[:] = ((_ovr_row(i) > _THR) & (jidx > kidx)).astype(jnp.float32)

        def _cond(c):
            return c[1]

        def _body(c):
            k, _ = c
            sup = lax.dot_general(m_s[:], k, (((1,), (0,)), ((), ())),
                                  preferred_element_type=jnp.float32,
                                  precision=_HI)
            nk = jnp.where(sup > 0.5, 0.0, alive)
            return nk, jnp.sum(jnp.abs(nk - k)) > 0.0

        keep_t, _ = lax.while_loop(_cond, _body, (alive, True))

        trow = lax.dot_general(keep_t, ident_s[:], (((0,), (0,)), ((), ())),
                               preferred_element_type=jnp.float32, precision=_HI)
        krows_s[pl.ds(i, 1), :] = trow
        prefix = lax.dot_general(trow, tri_s[:], (((1,), (0,)), ((), ())),
                                 preferred_element_type=jnp.float32,
                                 precision=_HI)
        tgt = jnp.where(trow > 0.5, cnt_s[0] + prefix - 1.0, float(_SCRAP) + gpos)
        tgt_out[pl.ds(i, 1), :] = tgt.astype(jnp.int32)
        cnt_s[0] = cnt_s[0] + jnp.sum(keep_t)

    @pl.when(done)
    def _skip():
        tgt_out[pl.ds(i, 1), :] = (float(_SCRAP) + gpos).astype(jnp.int32)


def _nms_targets(x1, y1, x2, y2):
    return pl.pallas_call(
        _nms_kernel,
        grid=(_T,),
        in_specs=[pl.BlockSpec((_T, _B), lambda i: (0, 0))] * 4,
        out_specs=pl.BlockSpec((_T, _B), lambda i: (0, 0)),
        out_shape=jax.ShapeDtypeStruct((_T, _B), jnp.int32),
        scratch_shapes=[
            pltpu.VMEM((_B, _B), jnp.float32),
            pltpu.VMEM((_B, _B), jnp.float32),
            pltpu.VMEM((_B, _B), jnp.float32),
            pltpu.VMEM((_T, _B), jnp.float32),
            pltpu.VMEM((_T, _B), jnp.float32),
            pltpu.SMEM((1,), jnp.float32),
        ],
    )(x1, y1, x2, y2)


def _sc_compact(props16, tgt):
    info = plsc.get_sparse_core_info()
    nw = info.num_cores * info.num_subcores
    per = _NPAD // nw
    mesh = plsc.VectorSubcoreMesh(core_axis_name="c", subcore_axis_name="s")

    @functools.partial(
        pl.kernel, mesh=mesh,
        compiler_params=pltpu.CompilerParams(use_tc_tiling_on_sc=False),
        out_type=jax.ShapeDtypeStruct((_BUFR, _ROWW), jnp.float32),
        scratch_types=[
            pltpu.VMEM((per,), jnp.int32),
            pltpu.VMEM((per, _ROWW), jnp.float32),
            pltpu.SemaphoreType.DMA,
        ],
    )
    def k(props_hbm, tgt_hbm, out_hbm, idx_v, rows_v, sem):
        wid = lax.axis_index("s") * info.num_cores + lax.axis_index("c")
        base = wid * per
        pltpu.sync_copy(tgt_hbm.at[pl.ds(base, per)], idx_v)
        pltpu.sync_copy(props_hbm.at[pl.ds(base, per)], rows_v)
        pltpu.async_copy(rows_v, out_hbm.at[idx_v], sem).wait()

    return k(props16, tgt)


def kernel(boxes, scores):
    x1 = jnp.clip(boxes[:, 0], 0.0, _IMW - 1.0)
    y1 = jnp.clip(boxes[:, 1], 0.0, _IMH - 1.0)
    x2 = jnp.clip(boxes[:, 2], 0.0, _IMW - 1.0)
    y2 = jnp.clip(boxes[:, 3], 0.0, _IMH - 1.0)
    ws = x2 - x1 + 1.0
    hs = y2 - y1 + 1.0
    size_ok = (ws >= _MIN) & (hs >= _MIN)
    sc = jnp.where(size_ok, scores, -jnp.inf)

    _, order = lax.sort((-sc, jnp.arange(_NB, dtype=jnp.int32)),
                        dimension=0, num_keys=1, is_stable=True)
    idx = order[:_NPAD]
    x1s = x1[idx]
    y1s = y1[idx]
    x2s = x2[idx]
    y2s = y2[idx]
    scs = sc[idx]

    tgt = _nms_targets(x1s.reshape(_T, _B), y1s.reshape(_T, _B),
                       x2s.reshape(_T, _B), y2s.reshape(_T, _B))

    props = jnp.concatenate(
        [x1s[:, None], y1s[:, None], x2s[:, None], y2s[:, None],
         scs[:, None], jnp.zeros((_NPAD, _ROWW - 5), jnp.float32)], axis=1)

    buf = _sc_compact(props, tgt.reshape(-1))

    cnt = jnp.sum((tgt.reshape(-1) < _POST).astype(jnp.int32))
    rows_ok = (jnp.arange(_POST) < cnt)[:, None]
    return jnp.where(rows_ok, buf[:_POST, :5], 0.0)
